# baseline (device time: 43747 ns/iter reference)
import functools

import jax
import jax.numpy as jnp
from jax import lax
from jax.experimental import pallas as pl
from jax.experimental.pallas import tpu as pltpu

N_DEV = 16
N_PLANE = 4
N_INPL = 4
NSLOT = 4


def kernel(x, w_mat):
    k_full, k_shard = x.shape
    n = w_mat.shape[1]
    m_blk = k_full // N_DEV

    def body(x_ref, w_hbm, out_ref, x_bf, z_buf, p_buf, w_buf,
             w_sems, send_sems, recv_z, recv_p):
        me = lax.axis_index("i")
        p = lax.div(me, N_INPL)
        q = lax.rem(me, N_INPL)

        def dev(plane, inpl):
            return (plane * N_INPL + inpl,)

        x_bf[...] = x_ref[...].astype(jnp.bfloat16).reshape(
            N_PLANE, N_INPL, m_blk, k_shard
        )

        order = [me]
        order += [p * N_INPL + lax.rem(q + d, N_INPL) for d in range(1, 4)]
        order += [lax.rem(p + e, N_PLANE) * N_INPL + q for e in range(1, 4)]
        for e in range(1, 4):
            order += [
                lax.rem(p + e, N_PLANE) * N_INPL + lax.rem(q + d, N_INPL)
                for d in range(1, 4)
            ]

        w_cps = {}

        def start_w(i):
            cp = pltpu.make_async_copy(
                w_hbm.at[pl.ds(order[i] * k_shard, k_shard), :],
                w_buf.at[i % NSLOT],
                w_sems.at[i % NSLOT],
            )
            cp.start()
            w_cps[i] = cp

        gemm_i = [0]

        def gemm(block):
            i = gemm_i[0]
            w_cps[i].wait()
            if i + 3 < N_DEV:
                start_w(i + 3)
            w_blk = w_buf[i % NSLOT].astype(jnp.bfloat16)
            acc = jnp.dot(block, w_blk, preferred_element_type=jnp.float32)
            if i == 0:
                out_ref[...] = acc
            else:
                out_ref[...] += acc
            gemm_i[0] = i + 1

        for i in range(3):
            start_w(i)

        barrier_sem = pltpu.get_barrier_semaphore()
        for e in range(1, 4):
            pl.semaphore_signal(
                barrier_sem, inc=1,
                device_id=dev(lax.rem(p + e, N_PLANE), q),
                device_id_type=pl.DeviceIdType.MESH,
            )
        for d in range(1, 4):
            pl.semaphore_signal(
                barrier_sem, inc=1,
                device_id=dev(p, lax.rem(q + d, N_INPL)),
                device_id_type=pl.DeviceIdType.MESH,
            )
        pl.semaphore_wait(barrier_sem, 6)

        sends = []

        for e in range(1, 4):
            e_r = 4 - e
            rdma = pltpu.make_async_remote_copy(
                src_ref=x_bf.at[lax.rem(p + e, N_PLANE)],
                dst_ref=z_buf.at[e_r - 1],
                send_sem=send_sems.at[e - 1],
                recv_sem=recv_z.at[e_r - 1],
                device_id=dev(lax.rem(p + e, N_PLANE), q),
                device_id_type=pl.DeviceIdType.MESH,
            )
            rdma.start()
            sends.append(rdma)

        for d in range(1, 4):
            j = lax.rem(q - d + N_INPL, N_INPL)
            rdma = pltpu.make_async_remote_copy(
                src_ref=x_bf.at[p, j],
                dst_ref=p_buf.at[d - 1, 0],
                send_sem=send_sems.at[3 + (d - 1)],
                recv_sem=recv_p.at[(d - 1) * 4],
                device_id=dev(p, j),
                device_id_type=pl.DeviceIdType.MESH,
            )
            rdma.start()
            sends.append(rdma)

        gemm(x_bf[p, q])

        for d in range(1, 4):
            recv = pltpu.make_async_remote_copy(
                src_ref=x_bf.at[p, q],
                dst_ref=p_buf.at[d - 1, 0],
                send_sem=send_sems.at[0],
                recv_sem=recv_p.at[(d - 1) * 4],
                device_id=dev(p, q),
                device_id_type=pl.DeviceIdType.MESH,
            )
            recv.wait_recv()
            gemm(p_buf[d - 1, 0])

        for e in range(1, 4):
            recv = pltpu.make_async_remote_copy(
                src_ref=x_bf.at[p],
                dst_ref=z_buf.at[e - 1],
                send_sem=send_sems.at[0],
                recv_sem=recv_z.at[e - 1],
                device_id=dev(p, q),
                device_id_type=pl.DeviceIdType.MESH,
            )
            recv.wait_recv()
            for d in range(1, 4):
                j = lax.rem(q - d + N_INPL, N_INPL)
                rdma = pltpu.make_async_remote_copy(
                    src_ref=z_buf.at[e - 1, j],
                    dst_ref=p_buf.at[d - 1, e],
                    send_sem=send_sems.at[6 + (e - 1) * 3 + (d - 1)],
                    recv_sem=recv_p.at[(d - 1) * 4 + e],
                    device_id=dev(p, j),
                    device_id_type=pl.DeviceIdType.MESH,
                )
                rdma.start()
                sends.append(rdma)
            gemm(z_buf[e - 1, q])

        for e in range(1, 4):
            for d in range(1, 4):
                recv = pltpu.make_async_remote_copy(
                    src_ref=x_bf.at[p, q],
                    dst_ref=p_buf.at[d - 1, e],
                    send_sem=send_sems.at[0],
                    recv_sem=recv_p.at[(d - 1) * 4 + e],
                    device_id=dev(p, q),
                    device_id_type=pl.DeviceIdType.MESH,
                )
                recv.wait_recv()
                gemm(p_buf[d - 1, e])

        for rdma in sends:
            rdma.wait_send()

        @functools.partial(
            pl.run_scoped, exit_sem=pltpu.SemaphoreType.REGULAR
        )
        def _(exit_sem):
            for e in range(1, 4):
                pl.semaphore_signal(
                    exit_sem, inc=1,
                    device_id=dev(lax.rem(p + e, N_PLANE), q),
                    device_id_type=pl.DeviceIdType.MESH,
                )
            for d in range(1, 4):
                pl.semaphore_signal(
                    exit_sem, inc=1,
                    device_id=dev(p, lax.rem(q + d, N_INPL)),
                    device_id_type=pl.DeviceIdType.MESH,
                )
            pl.semaphore_wait(exit_sem, 6)

    return pl.pallas_call(
        body,
        out_shape=jax.ShapeDtypeStruct((m_blk, n), jnp.float32),
        in_specs=[
            pl.BlockSpec(memory_space=pltpu.VMEM),
            pl.BlockSpec(memory_space=pl.ANY),
        ],
        out_specs=pl.BlockSpec(memory_space=pltpu.VMEM),
        scratch_shapes=[
            pltpu.VMEM((N_PLANE, N_INPL, m_blk, k_shard), jnp.bfloat16),
            pltpu.VMEM((3, N_INPL, m_blk, k_shard), jnp.bfloat16),
            pltpu.VMEM((3, 4, m_blk, k_shard), jnp.bfloat16),
            pltpu.VMEM((NSLOT, k_shard, n), jnp.float32),
            pltpu.SemaphoreType.DMA((NSLOT,)),
            pltpu.SemaphoreType.DMA((15,)),
            pltpu.SemaphoreType.DMA((3,)),
            pltpu.SemaphoreType.DMA((12,)),
        ],
        compiler_params=pltpu.CompilerParams(collective_id=0),
    )(x, w_mat)


# device time: 41119 ns/iter; 1.0639x vs baseline; 1.0639x over previous
import functools

import jax
import jax.numpy as jnp
from jax import lax
from jax.experimental import pallas as pl
from jax.experimental.pallas import tpu as pltpu

N_DEV = 16
N_PLANE = 4
N_INPL = 4
NSLOT = 6


def kernel(x, w_mat):
    k_full, k_shard = x.shape
    n = w_mat.shape[1]
    m_blk = k_full // N_DEV

    def body(x_ref, w_hbm, out_ref, x_bf, gather_ref, w_buf,
             w_sems, send_sems, recv_sems):
        me = lax.axis_index("i")
        p = lax.div(me, N_INPL)
        q = lax.rem(me, N_INPL)

        order = [me]
        order += [p * N_INPL + lax.rem(q + dd, N_INPL) for dd in range(1, 4)]
        for e in range(1, 4):
            order += [
                lax.rem(p + e, N_PLANE) * N_INPL + lax.rem(q + dd, N_INPL)
                for dd in range(4)
            ]

        x_bf[...] = x_ref[...].astype(jnp.bfloat16)

        w_cps = {}

        def start_w(i):
            cp = pltpu.make_async_copy(
                w_hbm.at[pl.ds(order[i] * k_shard, k_shard), :],
                w_buf.at[i % NSLOT],
                w_sems.at[i % NSLOT],
            )
            cp.start()
            w_cps[i] = cp

        gemm_i = [0]

        def gemm(block):
            i = gemm_i[0]
            w_cps[i].wait()
            if i + NSLOT - 1 < N_DEV:
                start_w(i + NSLOT - 1)
            w_blk = w_buf[i % NSLOT].astype(jnp.bfloat16)
            acc = jnp.dot(block, w_blk, preferred_element_type=jnp.float32)
            if i == 0:
                out_ref[...] = acc
            else:
                out_ref[...] += acc
            gemm_i[0] = i + 1

        for i in range(NSLOT - 1):
            start_w(i)

        barrier_sem = pltpu.get_barrier_semaphore()
        for d in range(1, N_DEV):
            pl.semaphore_signal(
                barrier_sem, inc=1,
                device_id=(lax.rem(me + d, N_DEV),),
                device_id_type=pl.DeviceIdType.MESH,
            )
        pl.semaphore_wait(barrier_sem, N_DEV - 1)

        sends = []

        def send_to(e, dd):
            tq = lax.rem(q + dd, N_INPL)
            tgt = lax.rem(p + e, N_PLANE) * N_INPL + tq
            e_r = (4 - e) % 4
            dd_r = (4 - dd) % 4
            slot = dd_r if e == 0 else 4 + (e_r - 1) * 4 + dd_r
            rdma = pltpu.make_async_remote_copy(
                src_ref=x_bf.at[pl.ds(tgt * m_blk, m_blk), :],
                dst_ref=gather_ref.at[slot],
                send_sem=send_sems.at[slot],
                recv_sem=recv_sems.at[slot],
                device_id=(tgt,),
                device_id_type=pl.DeviceIdType.MESH,
            )
            rdma.start()
            sends.append(rdma)

        for e in (2, 1, 3):
            for dd in range(4):
                send_to(e, dd)
        for dd in range(1, 4):
            send_to(0, dd)

        gemm(x_bf[pl.ds(me * m_blk, m_blk), :])

        for s in range(1, N_DEV):
            recv = pltpu.make_async_remote_copy(
                src_ref=x_bf.at[pl.ds(0, m_blk), :],
                dst_ref=gather_ref.at[s],
                send_sem=send_sems.at[0],
                recv_sem=recv_sems.at[s],
                device_id=(me,),
                device_id_type=pl.DeviceIdType.MESH,
            )
            recv.wait_recv()
            gemm(gather_ref[s])

        for rdma in sends:
            rdma.wait_send()

        @functools.partial(
            pl.run_scoped, exit_sem=pltpu.SemaphoreType.REGULAR
        )
        def _(exit_sem):
            for d in range(1, N_DEV):
                pl.semaphore_signal(
                    exit_sem, inc=1,
                    device_id=(lax.rem(me + d, N_DEV),),
                    device_id_type=pl.DeviceIdType.MESH,
                )
            pl.semaphore_wait(exit_sem, N_DEV - 1)

    return pl.pallas_call(
        body,
        out_shape=jax.ShapeDtypeStruct((m_blk, n), jnp.float32),
        in_specs=[
            pl.BlockSpec(memory_space=pltpu.VMEM),
            pl.BlockSpec(memory_space=pl.ANY),
        ],
        out_specs=pl.BlockSpec(memory_space=pltpu.VMEM),
        scratch_shapes=[
            pltpu.VMEM((k_full, k_shard), jnp.bfloat16),
            pltpu.VMEM((N_DEV, m_blk, k_shard), jnp.bfloat16),
            pltpu.VMEM((NSLOT, k_shard, n), jnp.float32),
            pltpu.SemaphoreType.DMA((NSLOT,)),
            pltpu.SemaphoreType.DMA((N_DEV,)),
            pltpu.SemaphoreType.DMA((N_DEV,)),
        ],
        compiler_params=pltpu.CompilerParams(collective_id=0),
    )(x, w_mat)


# device time: 39756 ns/iter; 1.1004x vs baseline; 1.0343x over previous
import functools

import jax
import jax.numpy as jnp
from jax import lax
from jax.experimental import pallas as pl
from jax.experimental.pallas import tpu as pltpu

N_DEV = 16
N_PLANE = 4
N_INPL = 4
NSLOT = 8


def kernel(x, w_mat):
    k_full, k_shard = x.shape
    n = w_mat.shape[1]
    m_blk = k_full // N_DEV

    def body(x_ref, w_hbm, out_ref, x_bf, gather_ref, w_buf,
             w_sems, send_sems, recv_sems):
        me = lax.axis_index("i")
        p = lax.div(me, N_INPL)
        q = lax.rem(me, N_INPL)

        order = [me]
        order += [p * N_INPL + lax.rem(q + dd, N_INPL) for dd in range(1, 4)]
        for e in range(1, 4):
            order += [
                lax.rem(p + e, N_PLANE) * N_INPL + lax.rem(q + dd, N_INPL)
                for dd in range(4)
            ]

        w_cps = {}

        def start_w(i):
            cp = pltpu.make_async_copy(
                w_hbm.at[pl.ds(order[i] * k_shard, k_shard), :],
                w_buf.at[i % NSLOT],
                w_sems.at[i % NSLOT],
            )
            cp.start()
            w_cps[i] = cp

        gemm_i = [0]

        def gemm(block):
            i = gemm_i[0]
            w_cps[i].wait()
            if i + NSLOT - 1 < N_DEV:
                start_w(i + NSLOT - 1)
            w_blk = w_buf[i % NSLOT].astype(jnp.bfloat16)
            acc = jnp.dot(block, w_blk, preferred_element_type=jnp.float32)
            if i == 0:
                out_ref[...] = acc
            else:
                out_ref[...] += acc
            gemm_i[0] = i + 1

        for i in range(NSLOT - 1):
            start_w(i)

        barrier_sem = pltpu.get_barrier_semaphore()
        for d in range(1, N_DEV):
            pl.semaphore_signal(
                barrier_sem, inc=1,
                device_id=(lax.rem(me + d, N_DEV),),
                device_id_type=pl.DeviceIdType.MESH,
            )
        x_bf[...] = x_ref[...].astype(jnp.bfloat16)
        pl.semaphore_wait(barrier_sem, N_DEV - 1)

        sends = []

        def send_to(e, dd):
            tq = lax.rem(q + dd, N_INPL)
            tgt = lax.rem(p + e, N_PLANE) * N_INPL + tq
            e_r = (4 - e) % 4
            dd_r = (4 - dd) % 4
            slot = dd_r if e == 0 else 4 + (e_r - 1) * 4 + dd_r
            rdma = pltpu.make_async_remote_copy(
                src_ref=x_bf.at[pl.ds(tgt * m_blk, m_blk), :],
                dst_ref=gather_ref.at[slot],
                send_sem=send_sems.at[slot],
                recv_sem=recv_sems.at[slot],
                device_id=(tgt,),
                device_id_type=pl.DeviceIdType.MESH,
            )
            rdma.start()
            sends.append(rdma)

        for e in (2, 1, 3):
            for dd in range(4):
                send_to(e, dd)
        for dd in range(1, 4):
            send_to(0, dd)

        gemm(x_bf[pl.ds(me * m_blk, m_blk), :])

        for s in range(1, N_DEV):
            recv = pltpu.make_async_remote_copy(
                src_ref=x_bf.at[pl.ds(0, m_blk), :],
                dst_ref=gather_ref.at[s],
                send_sem=send_sems.at[0],
                recv_sem=recv_sems.at[s],
                device_id=(me,),
                device_id_type=pl.DeviceIdType.MESH,
            )
            recv.wait_recv()
            gemm(gather_ref[s])

        for rdma in sends:
            rdma.wait_send()

        @functools.partial(
            pl.run_scoped, exit_sem=pltpu.SemaphoreType.REGULAR
        )
        def _(exit_sem):
            for d in range(1, N_DEV):
                pl.semaphore_signal(
                    exit_sem, inc=1,
                    device_id=(lax.rem(me + d, N_DEV),),
                    device_id_type=pl.DeviceIdType.MESH,
                )
            pl.semaphore_wait(exit_sem, N_DEV - 1)

    return pl.pallas_call(
        body,
        out_shape=jax.ShapeDtypeStruct((m_blk, n), jnp.float32),
        in_specs=[
            pl.BlockSpec(memory_space=pltpu.VMEM),
            pl.BlockSpec(memory_space=pl.ANY),
        ],
        out_specs=pl.BlockSpec(memory_space=pltpu.VMEM),
        scratch_shapes=[
            pltpu.VMEM((k_full, k_shard), jnp.bfloat16),
            pltpu.VMEM((N_DEV, m_blk, k_shard), jnp.bfloat16),
            pltpu.VMEM((NSLOT, k_shard, n), jnp.float32),
            pltpu.SemaphoreType.DMA((NSLOT,)),
            pltpu.SemaphoreType.DMA((N_DEV,)),
            pltpu.SemaphoreType.DMA((N_DEV,)),
        ],
        compiler_params=pltpu.CompilerParams(collective_id=0),
    )(x, w_mat)
